# baseline (speedup 1.0000x reference)
import functools
import jax
import jax.numpy as jnp
from jax import lax
from jax.experimental import pallas as pl
from jax.experimental.pallas import tpu as pltpu
from jax.experimental.pallas import tpu_sc as plsc

_N = 10000
_E = 320000
_F = 128
_ALPHA = 0.2

_NC = 2
_NS = 16
_NW = _NC * _NS
_EPT = _E // _NW
_CH = 80
_NCHUNK = _EPT // _CH
_NP = 10240
_RPT = _NP // _NS
_NJ = (_NCHUNK - 1) // 4


def _tc1_body(x_ref, w_ref, a1_ref, a2_ref, wh_ref, s1_ref, s2_ref):
    wh = jnp.dot(x_ref[...], w_ref[...],
                 preferred_element_type=jnp.float32,
                 precision=lax.Precision.HIGHEST)
    wh_ref[...] = wh
    s1_ref[...] = jnp.dot(wh, a1_ref[...],
                          preferred_element_type=jnp.float32,
                          precision=lax.Precision.HIGHEST)
    s2_ref[...] = jnp.dot(wh, a2_ref[...],
                          preferred_element_type=jnp.float32,
                          precision=lax.Precision.HIGHEST)


def _tc1(x, W, a1, a2):
    nb = 10
    rb = _N // nb
    return pl.pallas_call(
        _tc1_body,
        grid=(nb,),
        in_specs=[
            pl.BlockSpec((rb, _F), lambda i: (i, 0)),
            pl.BlockSpec((_F, _F), lambda i: (0, 0)),
            pl.BlockSpec((_F, 1), lambda i: (0, 0)),
            pl.BlockSpec((_F, 1), lambda i: (0, 0)),
        ],
        out_specs=[
            pl.BlockSpec((rb, _F), lambda i: (i, 0)),
            pl.BlockSpec((rb, 1), lambda i: (i, 0)),
            pl.BlockSpec((rb, 1), lambda i: (i, 0)),
        ],
        out_shape=[
            jax.ShapeDtypeStruct((_N, _F), jnp.float32),
            jax.ShapeDtypeStruct((_N, 1), jnp.float32),
            jax.ShapeDtypeStruct((_N, 1), jnp.float32),
        ],
    )(x, W, a1, a2)


def _sc_body(src_hbm, dst_hbm, s1_hbm, s2_hbm, wh_hbm, zero_hbm, zerod_hbm,
             out_hbm, outd_hbm,
             sidx, didx, s1g, s2g, rows, wrows, wcol, acc, dacc,
             semg, sems, semi):
    c = lax.axis_index("c")
    s = lax.axis_index("s")
    wid = c * _NS + s
    ebase = wid * _EPT

    def fire_idx(i, b, m):
        pltpu.async_copy(src_hbm.at[pl.ds(ebase + i * _CH, _CH)],
                         sidx.at[b], semi.at[b])
        pltpu.async_copy(dst_hbm.at[pl.ds(ebase + i * _CH, _CH)],
                         didx.at[m], semi.at[b])

    def wait_idx(b, m):
        pltpu.make_async_copy(src_hbm.at[pl.ds(0, _CH)],
                              sidx.at[b], semi.at[b]).wait()
        pltpu.make_async_copy(dst_hbm.at[pl.ds(0, _CH)],
                              didx.at[m], semi.at[b]).wait()

    def fire_gather(b, m):
        pltpu.async_copy(wh_hbm.at[sidx.at[b]], rows.at[b], semg.at[b])
        pltpu.async_copy(s1_hbm.at[sidx.at[b]], s1g.at[b], semg.at[b])
        pltpu.async_copy(s2_hbm.at[didx.at[m]], s2g.at[b], semg.at[b])

    def wait_gather(b, m):
        pltpu.make_async_copy(wh_hbm.at[sidx.at[b]], rows.at[b],
                              semg.at[b]).wait()
        pltpu.make_async_copy(s1_hbm.at[sidx.at[b]], s1g.at[b],
                              semg.at[b]).wait()
        pltpu.make_async_copy(s2_hbm.at[didx.at[m]], s2g.at[b],
                              semg.at[b]).wait()

    def fire_scatter(b, m):
        pltpu.async_copy(wrows.at[b], acc.at[didx.at[m]], sems.at[b],
                         add=True)
        pltpu.async_copy(wcol.at[b], dacc.at[didx.at[m]], sems.at[b],
                         add=True)

    def drain_scatter(b, m):
        pltpu.make_async_copy(wrows.at[b], acc.at[didx.at[m]],
                              sems.at[b]).wait()
        pltpu.make_async_copy(wcol.at[b], dacc.at[didx.at[m]],
                              sems.at[b]).wait()

    def process(b):
        @plsc.parallel_loop(0, _CH, step=16, unroll=5)
        def _(r0):
            z = s1g[b, pl.ds(r0, 16)] + s2g[b, pl.ds(r0, 16)]
            e = jnp.where(z > 0, z, z * _ALPHA)
            w = jnp.exp(e)
            wcol[b, pl.ds(r0, 16)] = w
            wvs = [jnp.full((16,), w[rl], jnp.float32) for rl in range(16)]
            for jj in range(8):
                for rl in range(16):
                    wrows[b, r0 + rl, pl.ds(jj * 16, 16)] = (
                        rows[b, r0 + rl, pl.ds(jj * 16, 16)] * wvs[rl])

    pltpu.sync_copy(zero_hbm.at[pl.ds(s * _RPT, _RPT)],
                    acc.at[pl.ds(s * _RPT, _RPT)])
    pltpu.sync_copy(zerod_hbm.at[pl.ds(s * _RPT, _RPT)],
                    dacc.at[pl.ds(s * _RPT, _RPT)])
    fire_idx(0, 0, 0)
    fire_idx(1, 1, 1)
    wait_idx(0, 0)
    fire_gather(0, 0)
    plsc.subcore_barrier()

    def body(j, carry):
        for k in range(4):
            b = k & 1
            i = 4 * j + k
            wait_gather(b, k)
            wait_idx(1 - b, (k + 1) & 3)
            if k < 2:
                @pl.when(j > 0)
                def _():
                    drain_scatter(b, (k + 2) & 3)
            else:
                drain_scatter(b, (k + 2) & 3)
            if k == 3:
                @pl.when(j < _NJ - 1)
                def _():
                    fire_idx(i + 2, b, (k + 2) & 3)
            else:
                fire_idx(i + 2, b, (k + 2) & 3)
            fire_gather(1 - b, (k + 1) & 3)
            process(b)
            fire_scatter(b, k)
        return carry

    lax.fori_loop(0, _NJ, body, 0)

    wait_gather(0, 0)
    drain_scatter(0, 2)
    process(0)
    fire_scatter(0, 0)
    drain_scatter(1, 3)
    drain_scatter(0, 0)

    plsc.subcore_barrier()
    pltpu.sync_copy(acc.at[pl.ds(s * _RPT, _RPT)],
                    out_hbm.at[c, pl.ds(s * _RPT, _RPT)])
    pltpu.sync_copy(dacc.at[pl.ds(s * _RPT, _RPT)],
                    outd_hbm.at[c, pl.ds(s * _RPT, _RPT)])


_sc_call = functools.partial(
    pl.kernel,
    out_type=(jax.ShapeDtypeStruct((_NC, _NP, _F), jnp.float32),
              jax.ShapeDtypeStruct((_NC, _NP), jnp.float32)),
    mesh=plsc.VectorSubcoreMesh(core_axis_name="c", subcore_axis_name="s",
                                num_cores=_NC, num_subcores=_NS),
    scratch_types=[
        pltpu.VMEM((2, _CH), jnp.int32),
        pltpu.VMEM((4, _CH), jnp.int32),
        pltpu.VMEM((2, _CH), jnp.float32),
        pltpu.VMEM((2, _CH), jnp.float32),
        pltpu.VMEM((2, _CH, _F), jnp.float32),
        pltpu.VMEM((2, _CH, _F), jnp.float32),
        pltpu.VMEM((2, _CH), jnp.float32),
        pltpu.VMEM_SHARED((_NP, _F), jnp.float32),
        pltpu.VMEM_SHARED((_NP,), jnp.float32),
        pltpu.SemaphoreType.DMA((2,)),
        pltpu.SemaphoreType.DMA((2,)),
        pltpu.SemaphoreType.DMA((2,)),
    ],
    compiler_params=pltpu.CompilerParams(use_tc_tiling_on_sc=False,
                                         needs_layout_passes=False),
)(_sc_body)


def _tc2_body(acc_ref, dacc_ref, x_ref, wih_ref, whh_ref, bih_ref, bhh_ref,
              out_ref):
    h_msg = acc_ref[0] + acc_ref[1]
    denom = dacc_ref[0] + dacc_ref[1]
    h_prime = h_msg / jnp.maximum(denom, 1e-9)
    gi = jnp.dot(x_ref[...], wih_ref[...],
                 preferred_element_type=jnp.float32,
                 precision=lax.Precision.HIGHEST) + bih_ref[...]
    gh = jnp.dot(h_prime, whh_ref[...],
                 preferred_element_type=jnp.float32,
                 precision=lax.Precision.HIGHEST) + bhh_ref[...]
    r = jax.nn.sigmoid(gi[:, :_F] + gh[:, :_F])
    z = jax.nn.sigmoid(gi[:, _F:2 * _F] + gh[:, _F:2 * _F])
    n = jnp.tanh(gi[:, 2 * _F:] + r * gh[:, 2 * _F:])
    out_ref[...] = (1.0 - z) * n + z * h_prime


def _tc2(acc, dacc, x, wihT, whhT, bih, bhh):
    nb = 10
    rb = _N // nb
    return pl.pallas_call(
        _tc2_body,
        grid=(nb,),
        in_specs=[
            pl.BlockSpec((_NC, rb, _F), lambda i: (0, i, 0)),
            pl.BlockSpec((_NC, rb, 1), lambda i: (0, i, 0)),
            pl.BlockSpec((rb, _F), lambda i: (i, 0)),
            pl.BlockSpec((_F, 3 * _F), lambda i: (0, 0)),
            pl.BlockSpec((_F, 3 * _F), lambda i: (0, 0)),
            pl.BlockSpec((1, 3 * _F), lambda i: (0, 0)),
            pl.BlockSpec((1, 3 * _F), lambda i: (0, 0)),
        ],
        out_specs=pl.BlockSpec((rb, _F), lambda i: (i, 0)),
        out_shape=jax.ShapeDtypeStruct((_N, _F), jnp.float32),
    )(acc, dacc, x, wihT, whhT, bih, bhh)


def kernel(x, edge_index, W, a, W_ih, W_hh, b_ih, b_hh):
    src = edge_index[0]
    dst = edge_index[1]
    a1 = a[:_F]
    a2 = a[_F:]
    wh, s1, s2 = _tc1(x, W, a1, a2)
    zeros = jnp.zeros((_NP, _F), jnp.float32)
    zerosd = jnp.zeros((_NP,), jnp.float32)
    acc, dacc = _sc_call(src, dst, s1.reshape(_N), s2.reshape(_N), wh,
                         zeros, zerosd)
    return _tc2(acc, dacc.reshape(_NC, _NP, 1), x, W_ih.T, W_hh.T,
                b_ih.reshape(1, 3 * _F), b_hh.reshape(1, 3 * _F))


# speedup vs baseline: 1.0545x; 1.0545x over previous
import functools
import jax
import jax.numpy as jnp
from jax import lax
from jax.experimental import pallas as pl
from jax.experimental.pallas import tpu as pltpu
from jax.experimental.pallas import tpu_sc as plsc

_N = 10000
_E = 320000
_F = 128
_ALPHA = 0.2

_NC = 2
_NS = 16
_NW = _NC * _NS
_EPT = _E // _NW
_CH = 80
_NCHUNK = _EPT // _CH
_NP = 10240
_RPT = _NP // _NS
_NJ = (_NCHUNK - 1) // 4


def _tc1_body(x_ref, w_ref, a1_ref, a2_ref, wh_ref, s1_ref, s2_ref):
    wh = jnp.dot(x_ref[...], w_ref[...],
                 preferred_element_type=jnp.float32,
                 precision=lax.Precision.HIGHEST)
    wh_ref[...] = wh
    s1_ref[...] = jnp.dot(wh, a1_ref[...],
                          preferred_element_type=jnp.float32,
                          precision=lax.Precision.HIGHEST)
    s2_ref[...] = jnp.dot(wh, a2_ref[...],
                          preferred_element_type=jnp.float32,
                          precision=lax.Precision.HIGHEST)


def _tc1(x, W, a1, a2):
    nb = 10
    rb = _N // nb
    return pl.pallas_call(
        _tc1_body,
        grid=(nb,),
        in_specs=[
            pl.BlockSpec((rb, _F), lambda i: (i, 0)),
            pl.BlockSpec((_F, _F), lambda i: (0, 0)),
            pl.BlockSpec((_F, 1), lambda i: (0, 0)),
            pl.BlockSpec((_F, 1), lambda i: (0, 0)),
        ],
        out_specs=[
            pl.BlockSpec((rb, _F), lambda i: (i, 0)),
            pl.BlockSpec((rb, 1), lambda i: (i, 0)),
            pl.BlockSpec((rb, 1), lambda i: (i, 0)),
        ],
        out_shape=[
            jax.ShapeDtypeStruct((_N, _F), jnp.float32),
            jax.ShapeDtypeStruct((_N, 1), jnp.float32),
            jax.ShapeDtypeStruct((_N, 1), jnp.float32),
        ],
    )(x, W, a1, a2)


def _sc_body(src_hbm, dst_hbm, s1_hbm, s2_hbm, wh_hbm, zero_hbm, zerod_hbm,
             out_hbm, outd_hbm,
             sidx, didx, s1g, s2g, rows, wrows, wcol, acc, dacc,
             semg, sems, semi):
    c = lax.axis_index("c")
    s = lax.axis_index("s")
    wid = c * _NS + s
    ebase = wid * _EPT

    def fire_idx(i, b, m):
        pltpu.async_copy(src_hbm.at[pl.ds(ebase + i * _CH, _CH)],
                         sidx.at[b], semi.at[b])
        pltpu.async_copy(dst_hbm.at[pl.ds(ebase + i * _CH, _CH)],
                         didx.at[m], semi.at[b])

    def wait_idx(b, m):
        pltpu.make_async_copy(src_hbm.at[pl.ds(0, _CH)],
                              sidx.at[b], semi.at[b]).wait()
        pltpu.make_async_copy(dst_hbm.at[pl.ds(0, _CH)],
                              didx.at[m], semi.at[b]).wait()

    def fire_gather(b, m):
        pltpu.async_copy(wh_hbm.at[sidx.at[b]], rows.at[b], semg.at[b])
        pltpu.async_copy(s1_hbm.at[sidx.at[b]], s1g.at[b], semg.at[b])
        pltpu.async_copy(s2_hbm.at[didx.at[m]], s2g.at[b], semg.at[b])

    def wait_gather(b, m):
        pltpu.make_async_copy(wh_hbm.at[sidx.at[b]], rows.at[b],
                              semg.at[b]).wait()
        pltpu.make_async_copy(s1_hbm.at[sidx.at[b]], s1g.at[b],
                              semg.at[b]).wait()
        pltpu.make_async_copy(s2_hbm.at[didx.at[m]], s2g.at[b],
                              semg.at[b]).wait()

    def fire_scatter(b, m):
        pltpu.async_copy(wrows.at[b], acc.at[didx.at[m]], sems.at[b],
                         add=True)
        pltpu.async_copy(wcol.at[b], dacc.at[didx.at[m]], sems.at[b],
                         add=True)

    def drain_scatter(b, m):
        pltpu.make_async_copy(wrows.at[b], acc.at[didx.at[m]],
                              sems.at[b]).wait()
        pltpu.make_async_copy(wcol.at[b], dacc.at[didx.at[m]],
                              sems.at[b]).wait()

    def process(b):
        @plsc.parallel_loop(0, _CH, step=16, unroll=1)
        def _(r0):
            z = s1g[b, pl.ds(r0, 16)] + s2g[b, pl.ds(r0, 16)]
            e = jnp.where(z > 0, z, z * _ALPHA)
            w = jnp.exp(e)
            wcol[b, pl.ds(r0, 16)] = w
            wvs = [jnp.full((16,), w[rl], jnp.float32) for rl in range(16)]
            for jj in range(8):
                for rl in range(16):
                    wrows[b, r0 + rl, pl.ds(jj * 16, 16)] = (
                        rows[b, r0 + rl, pl.ds(jj * 16, 16)] * wvs[rl])

    pltpu.sync_copy(zero_hbm.at[pl.ds(s * _RPT, _RPT)],
                    acc.at[pl.ds(s * _RPT, _RPT)])
    pltpu.sync_copy(zerod_hbm.at[pl.ds(s * _RPT, _RPT)],
                    dacc.at[pl.ds(s * _RPT, _RPT)])
    fire_idx(0, 0, 0)
    fire_idx(1, 1, 1)
    wait_idx(0, 0)
    fire_gather(0, 0)
    plsc.subcore_barrier()

    def body(j, carry):
        for k in range(4):
            b = k & 1
            i = 4 * j + k
            wait_gather(b, k)
            wait_idx(1 - b, (k + 1) & 3)
            if k < 2:
                @pl.when(j > 0)
                def _():
                    drain_scatter(b, (k + 2) & 3)
            else:
                drain_scatter(b, (k + 2) & 3)
            if k == 3:
                @pl.when(j < _NJ - 1)
                def _():
                    fire_idx(i + 2, b, (k + 2) & 3)
            else:
                fire_idx(i + 2, b, (k + 2) & 3)
            fire_gather(1 - b, (k + 1) & 3)
            process(b)
            fire_scatter(b, k)
        return carry

    lax.fori_loop(0, _NJ, body, 0)

    wait_gather(0, 0)
    drain_scatter(0, 2)
    process(0)
    fire_scatter(0, 0)
    drain_scatter(1, 3)
    drain_scatter(0, 0)

    plsc.subcore_barrier()
    pltpu.sync_copy(acc.at[pl.ds(s * _RPT, _RPT)],
                    out_hbm.at[c, pl.ds(s * _RPT, _RPT)])
    pltpu.sync_copy(dacc.at[pl.ds(s * _RPT, _RPT)],
                    outd_hbm.at[c, pl.ds(s * _RPT, _RPT)])


_sc_call = functools.partial(
    pl.kernel,
    out_type=(jax.ShapeDtypeStruct((_NC, _NP, _F), jnp.float32),
              jax.ShapeDtypeStruct((_NC, _NP), jnp.float32)),
    mesh=plsc.VectorSubcoreMesh(core_axis_name="c", subcore_axis_name="s",
                                num_cores=_NC, num_subcores=_NS),
    scratch_types=[
        pltpu.VMEM((2, _CH), jnp.int32),
        pltpu.VMEM((4, _CH), jnp.int32),
        pltpu.VMEM((2, _CH), jnp.float32),
        pltpu.VMEM((2, _CH), jnp.float32),
        pltpu.VMEM((2, _CH, _F), jnp.float32),
        pltpu.VMEM((2, _CH, _F), jnp.float32),
        pltpu.VMEM((2, _CH), jnp.float32),
        pltpu.VMEM_SHARED((_NP, _F), jnp.float32),
        pltpu.VMEM_SHARED((_NP,), jnp.float32),
        pltpu.SemaphoreType.DMA((2,)),
        pltpu.SemaphoreType.DMA((2,)),
        pltpu.SemaphoreType.DMA((2,)),
    ],
    compiler_params=pltpu.CompilerParams(use_tc_tiling_on_sc=False,
                                         needs_layout_passes=False),
)(_sc_body)


def _tc2_body(acc_ref, dacc_ref, x_ref, wih_ref, whh_ref, bih_ref, bhh_ref,
              out_ref):
    h_msg = acc_ref[0] + acc_ref[1]
    denom = dacc_ref[0] + dacc_ref[1]
    h_prime = h_msg / jnp.maximum(denom, 1e-9)
    gi = jnp.dot(x_ref[...], wih_ref[...],
                 preferred_element_type=jnp.float32,
                 precision=lax.Precision.HIGHEST) + bih_ref[...]
    gh = jnp.dot(h_prime, whh_ref[...],
                 preferred_element_type=jnp.float32,
                 precision=lax.Precision.HIGHEST) + bhh_ref[...]
    r = jax.nn.sigmoid(gi[:, :_F] + gh[:, :_F])
    z = jax.nn.sigmoid(gi[:, _F:2 * _F] + gh[:, _F:2 * _F])
    n = jnp.tanh(gi[:, 2 * _F:] + r * gh[:, 2 * _F:])
    out_ref[...] = (1.0 - z) * n + z * h_prime


def _tc2(acc, dacc, x, wihT, whhT, bih, bhh):
    nb = 10
    rb = _N // nb
    return pl.pallas_call(
        _tc2_body,
        grid=(nb,),
        in_specs=[
            pl.BlockSpec((_NC, rb, _F), lambda i: (0, i, 0)),
            pl.BlockSpec((_NC, rb, 1), lambda i: (0, i, 0)),
            pl.BlockSpec((rb, _F), lambda i: (i, 0)),
            pl.BlockSpec((_F, 3 * _F), lambda i: (0, 0)),
            pl.BlockSpec((_F, 3 * _F), lambda i: (0, 0)),
            pl.BlockSpec((1, 3 * _F), lambda i: (0, 0)),
            pl.BlockSpec((1, 3 * _F), lambda i: (0, 0)),
        ],
        out_specs=pl.BlockSpec((rb, _F), lambda i: (i, 0)),
        out_shape=jax.ShapeDtypeStruct((_N, _F), jnp.float32),
    )(acc, dacc, x, wihT, whhT, bih, bhh)


def kernel(x, edge_index, W, a, W_ih, W_hh, b_ih, b_hh):
    src = edge_index[0]
    dst = edge_index[1]
    a1 = a[:_F]
    a2 = a[_F:]
    wh, s1, s2 = _tc1(x, W, a1, a2)
    zeros = jnp.zeros((_NP, _F), jnp.float32)
    zerosd = jnp.zeros((_NP,), jnp.float32)
    acc, dacc = _sc_call(src, dst, s1.reshape(_N), s2.reshape(_N), wh,
                         zeros, zerosd)
    return _tc2(acc, dacc.reshape(_NC, _NP, 1), x, W_ih.T, W_hh.T,
                b_ih.reshape(1, 3 * _F), b_hh.reshape(1, 3 * _F))


# speedup vs baseline: 1.0566x; 1.0019x over previous
import functools
import jax
import jax.numpy as jnp
from jax import lax
from jax.experimental import pallas as pl
from jax.experimental.pallas import tpu as pltpu
from jax.experimental.pallas import tpu_sc as plsc

_N = 10000
_E = 320000
_F = 128
_ALPHA = 0.2

_NC = 2
_NS = 16
_NW = _NC * _NS
_EPT = _E // _NW
_CH = 80
_NCHUNK = _EPT // _CH
_NP = 10240
_RPT = _NP // _NS
_NJ = (_NCHUNK - 1) // 4


def _tc1_body(x_ref, w_ref, a1_ref, a2_ref, wh_ref, s1_ref, s2_ref):
    wh = jnp.dot(x_ref[...], w_ref[...],
                 preferred_element_type=jnp.float32,
                 precision=lax.Precision.HIGHEST)
    wh_ref[...] = wh
    s1_ref[...] = jnp.dot(wh, a1_ref[...],
                          preferred_element_type=jnp.float32,
                          precision=lax.Precision.HIGHEST)
    s2_ref[...] = jnp.dot(wh, a2_ref[...],
                          preferred_element_type=jnp.float32,
                          precision=lax.Precision.HIGHEST)


def _tc1(x, W, a1, a2):
    nb = 10
    rb = _N // nb
    return pl.pallas_call(
        _tc1_body,
        grid=(nb,),
        in_specs=[
            pl.BlockSpec((rb, _F), lambda i: (i, 0)),
            pl.BlockSpec((_F, _F), lambda i: (0, 0)),
            pl.BlockSpec((_F, 1), lambda i: (0, 0)),
            pl.BlockSpec((_F, 1), lambda i: (0, 0)),
        ],
        out_specs=[
            pl.BlockSpec((rb, _F), lambda i: (i, 0)),
            pl.BlockSpec((rb, 1), lambda i: (i, 0)),
            pl.BlockSpec((rb, 1), lambda i: (i, 0)),
        ],
        out_shape=[
            jax.ShapeDtypeStruct((_N, _F), jnp.float32),
            jax.ShapeDtypeStruct((_N, 1), jnp.float32),
            jax.ShapeDtypeStruct((_N, 1), jnp.float32),
        ],
    )(x, W, a1, a2)


def _sc_body(src_hbm, dst_hbm, s1_hbm, s2_hbm, wh_hbm, zero_hbm, zerod_hbm,
             out_hbm, outd_hbm,
             sidx, didx, s1g, s2g, rows, wrows, wcol, acc, dacc,
             semg, sems, semi):
    c = lax.axis_index("c")
    s = lax.axis_index("s")
    wid = c * _NS + s
    ebase = wid * _EPT

    def fire_idx(i, b, m):
        pltpu.async_copy(src_hbm.at[pl.ds(ebase + i * _CH, _CH)],
                         sidx.at[b], semi.at[b])
        pltpu.async_copy(dst_hbm.at[pl.ds(ebase + i * _CH, _CH)],
                         didx.at[m], semi.at[b])

    def wait_idx(b, m):
        pltpu.make_async_copy(src_hbm.at[pl.ds(0, _CH)],
                              sidx.at[b], semi.at[b]).wait()
        pltpu.make_async_copy(dst_hbm.at[pl.ds(0, _CH)],
                              didx.at[m], semi.at[b]).wait()

    def fire_gather(b, m):
        pltpu.async_copy(wh_hbm.at[sidx.at[b]], rows.at[b], semg.at[b])
        pltpu.async_copy(s1_hbm.at[sidx.at[b]], s1g.at[b], semg.at[b])
        pltpu.async_copy(s2_hbm.at[didx.at[m]], s2g.at[b], semg.at[b])

    def wait_gather(b, m):
        pltpu.make_async_copy(wh_hbm.at[sidx.at[b]], rows.at[b],
                              semg.at[b]).wait()
        pltpu.make_async_copy(s1_hbm.at[sidx.at[b]], s1g.at[b],
                              semg.at[b]).wait()
        pltpu.make_async_copy(s2_hbm.at[didx.at[m]], s2g.at[b],
                              semg.at[b]).wait()

    def fire_scatter(b, m):
        pltpu.async_copy(wrows.at[b], acc.at[didx.at[m]], sems.at[b],
                         add=True)
        pltpu.async_copy(wcol.at[b], dacc.at[didx.at[m]], sems.at[b],
                         add=True)

    def drain_scatter(b, m):
        pltpu.make_async_copy(wrows.at[b], acc.at[didx.at[m]],
                              sems.at[b]).wait()
        pltpu.make_async_copy(wcol.at[b], dacc.at[didx.at[m]],
                              sems.at[b]).wait()

    def process(b):
        @plsc.parallel_loop(0, _CH, step=16, unroll=1)
        def _(r0):
            z = s1g[b, pl.ds(r0, 16)] + s2g[b, pl.ds(r0, 16)]
            e = jnp.where(z > 0, z, z * _ALPHA)
            w = jnp.exp(e)
            wcol[b, pl.ds(r0, 16)] = w
            wvs = [jnp.full((16,), w[rl], jnp.float32) for rl in range(16)]
            for jj in range(8):
                for rl in range(16):
                    wrows[b, r0 + rl, pl.ds(jj * 16, 16)] = (
                        rows[b, r0 + rl, pl.ds(jj * 16, 16)] * wvs[rl])

    pltpu.sync_copy(zero_hbm.at[pl.ds(s * _RPT, _RPT)],
                    acc.at[pl.ds(s * _RPT, _RPT)])
    pltpu.sync_copy(zerod_hbm.at[pl.ds(s * _RPT, _RPT)],
                    dacc.at[pl.ds(s * _RPT, _RPT)])
    fire_idx(0, 0, 0)
    fire_idx(1, 1, 1)
    wait_idx(0, 0)
    fire_gather(0, 0)
    plsc.subcore_barrier()

    def body(j, carry):
        for k in range(4):
            b = k & 1
            i = 4 * j + k
            with jax.named_scope("wgat"):
                wait_gather(b, k)
            with jax.named_scope("widx"):
                wait_idx(1 - b, (k + 1) & 3)
            if k < 2:
                @pl.when(j > 0)
                def _():
                    drain_scatter(b, (k + 2) & 3)
            else:
                drain_scatter(b, (k + 2) & 3)
            if k == 3:
                @pl.when(j < _NJ - 1)
                def _():
                    fire_idx(i + 2, b, (k + 2) & 3)
            else:
                fire_idx(i + 2, b, (k + 2) & 3)
            fire_gather(1 - b, (k + 1) & 3)
            with jax.named_scope("proc"):
                process(b)
            with jax.named_scope("fsca"):
                fire_scatter(b, k)
        return carry

    lax.fori_loop(0, _NJ, body, 0)

    wait_gather(0, 0)
    drain_scatter(0, 2)
    process(0)
    fire_scatter(0, 0)
    drain_scatter(1, 3)
    drain_scatter(0, 0)

    plsc.subcore_barrier()
    pltpu.sync_copy(acc.at[pl.ds(s * _RPT, _RPT)],
                    out_hbm.at[c, pl.ds(s * _RPT, _RPT)])
    pltpu.sync_copy(dacc.at[pl.ds(s * _RPT, _RPT)],
                    outd_hbm.at[c, pl.ds(s * _RPT, _RPT)])


_sc_call = functools.partial(
    pl.kernel,
    out_type=(jax.ShapeDtypeStruct((_NC, _NP, _F), jnp.float32),
              jax.ShapeDtypeStruct((_NC, _NP), jnp.float32)),
    mesh=plsc.VectorSubcoreMesh(core_axis_name="c", subcore_axis_name="s",
                                num_cores=_NC, num_subcores=_NS),
    scratch_types=[
        pltpu.VMEM((2, _CH), jnp.int32),
        pltpu.VMEM((4, _CH), jnp.int32),
        pltpu.VMEM((2, _CH), jnp.float32),
        pltpu.VMEM((2, _CH), jnp.float32),
        pltpu.VMEM((2, _CH, _F), jnp.float32),
        pltpu.VMEM((2, _CH, _F), jnp.float32),
        pltpu.VMEM((2, _CH), jnp.float32),
        pltpu.VMEM_SHARED((_NP, _F), jnp.float32),
        pltpu.VMEM_SHARED((_NP,), jnp.float32),
        pltpu.SemaphoreType.DMA((2,)),
        pltpu.SemaphoreType.DMA((2,)),
        pltpu.SemaphoreType.DMA((2,)),
    ],
    compiler_params=pltpu.CompilerParams(use_tc_tiling_on_sc=False,
                                         needs_layout_passes=False),
)(_sc_body)


def _tc2_body(acc_ref, dacc_ref, x_ref, wih_ref, whh_ref, bih_ref, bhh_ref,
              out_ref):
    h_msg = acc_ref[0] + acc_ref[1]
    denom = dacc_ref[0] + dacc_ref[1]
    h_prime = h_msg / jnp.maximum(denom, 1e-9)
    gi = jnp.dot(x_ref[...], wih_ref[...],
                 preferred_element_type=jnp.float32,
                 precision=lax.Precision.HIGHEST) + bih_ref[...]
    gh = jnp.dot(h_prime, whh_ref[...],
                 preferred_element_type=jnp.float32,
                 precision=lax.Precision.HIGHEST) + bhh_ref[...]
    r = jax.nn.sigmoid(gi[:, :_F] + gh[:, :_F])
    z = jax.nn.sigmoid(gi[:, _F:2 * _F] + gh[:, _F:2 * _F])
    n = jnp.tanh(gi[:, 2 * _F:] + r * gh[:, 2 * _F:])
    out_ref[...] = (1.0 - z) * n + z * h_prime


def _tc2(acc, dacc, x, wihT, whhT, bih, bhh):
    nb = 10
    rb = _N // nb
    return pl.pallas_call(
        _tc2_body,
        grid=(nb,),
        in_specs=[
            pl.BlockSpec((_NC, rb, _F), lambda i: (0, i, 0)),
            pl.BlockSpec((_NC, rb, 1), lambda i: (0, i, 0)),
            pl.BlockSpec((rb, _F), lambda i: (i, 0)),
            pl.BlockSpec((_F, 3 * _F), lambda i: (0, 0)),
            pl.BlockSpec((_F, 3 * _F), lambda i: (0, 0)),
            pl.BlockSpec((1, 3 * _F), lambda i: (0, 0)),
            pl.BlockSpec((1, 3 * _F), lambda i: (0, 0)),
        ],
        out_specs=pl.BlockSpec((rb, _F), lambda i: (i, 0)),
        out_shape=jax.ShapeDtypeStruct((_N, _F), jnp.float32),
    )(acc, dacc, x, wihT, whhT, bih, bhh)


def kernel(x, edge_index, W, a, W_ih, W_hh, b_ih, b_hh):
    src = edge_index[0]
    dst = edge_index[1]
    a1 = a[:_F]
    a2 = a[_F:]
    wh, s1, s2 = _tc1(x, W, a1, a2)
    zeros = jnp.zeros((_NP, _F), jnp.float32)
    zerosd = jnp.zeros((_NP,), jnp.float32)
    acc, dacc = _sc_call(src, dst, s1.reshape(_N), s2.reshape(_N), wh,
                         zeros, zerosd)
    return _tc2(acc, dacc.reshape(_NC, _NP, 1), x, W_ih.T, W_hh.T,
                b_ih.reshape(1, 3 * _F), b_hh.reshape(1, 3 * _F))


# speedup vs baseline: 1.2256x; 1.1600x over previous
import functools
import jax
import jax.numpy as jnp
from jax import lax
from jax.experimental import pallas as pl
from jax.experimental.pallas import tpu as pltpu
from jax.experimental.pallas import tpu_sc as plsc

_N = 10000
_E = 320000
_F = 128
_ALPHA = 0.2

_NC = 2
_NS = 16
_NW = _NC * _NS
_EPT = _E // _NW
_CH = 80
_NCHUNK = _EPT // _CH
_NP = 10240
_RPT = _NP // _NS
_NJ = (_NCHUNK - 1) // 4


def _tc1_body(x_ref, w_ref, a1_ref, a2_ref, wh_ref, s1_ref, s2_ref):
    wh = jnp.dot(x_ref[...], w_ref[...],
                 preferred_element_type=jnp.float32,
                 precision=lax.Precision.HIGHEST)
    wh_ref[...] = wh
    s1_ref[...] = jnp.dot(wh, a1_ref[...],
                          preferred_element_type=jnp.float32,
                          precision=lax.Precision.HIGHEST)
    s2_ref[...] = jnp.dot(wh, a2_ref[...],
                          preferred_element_type=jnp.float32,
                          precision=lax.Precision.HIGHEST)


def _tc1(x, W, a1, a2):
    nb = 10
    rb = _N // nb
    return pl.pallas_call(
        _tc1_body,
        grid=(nb,),
        in_specs=[
            pl.BlockSpec((rb, _F), lambda i: (i, 0)),
            pl.BlockSpec((_F, _F), lambda i: (0, 0)),
            pl.BlockSpec((_F, 1), lambda i: (0, 0)),
            pl.BlockSpec((_F, 1), lambda i: (0, 0)),
        ],
        out_specs=[
            pl.BlockSpec((rb, _F), lambda i: (i, 0)),
            pl.BlockSpec((rb, 1), lambda i: (i, 0)),
            pl.BlockSpec((rb, 1), lambda i: (i, 0)),
        ],
        out_shape=[
            jax.ShapeDtypeStruct((_N, _F), jnp.float32),
            jax.ShapeDtypeStruct((_N, 1), jnp.float32),
            jax.ShapeDtypeStruct((_N, 1), jnp.float32),
        ],
    )(x, W, a1, a2)


def _sc_body(src_hbm, dst_hbm, s1_hbm, s2_hbm, wh_hbm, zero_hbm, zerod_hbm,
             out_hbm, outd_hbm,
             sidx, didx, s1g, s2g, rows, wrows, wcol, acc, dacc,
             semg, sems, semi):
    c = lax.axis_index("c")
    s = lax.axis_index("s")
    wid = c * _NS + s
    ebase = wid * _EPT

    def fire_idx(i, b, m):
        pltpu.async_copy(src_hbm.at[pl.ds(ebase + i * _CH, _CH)],
                         sidx.at[m], semi.at[b])
        pltpu.async_copy(dst_hbm.at[pl.ds(ebase + i * _CH, _CH)],
                         didx.at[m], semi.at[b])

    def wait_idx(b, m):
        pltpu.make_async_copy(src_hbm.at[pl.ds(0, _CH)],
                              sidx.at[m], semi.at[b]).wait()
        pltpu.make_async_copy(dst_hbm.at[pl.ds(0, _CH)],
                              didx.at[m], semi.at[b]).wait()

    def fire_gather(b, m):
        pltpu.async_copy(wh_hbm.at[sidx.at[m]], rows.at[b], semg.at[b])
        pltpu.async_copy(s1_hbm.at[sidx.at[m]], s1g.at[b], semg.at[b])
        pltpu.async_copy(s2_hbm.at[didx.at[m]], s2g.at[b], semg.at[b])

    def wait_gather(b, m):
        pltpu.make_async_copy(wh_hbm.at[sidx.at[m]], rows.at[b],
                              semg.at[b]).wait()
        pltpu.make_async_copy(s1_hbm.at[sidx.at[m]], s1g.at[b],
                              semg.at[b]).wait()
        pltpu.make_async_copy(s2_hbm.at[didx.at[m]], s2g.at[b],
                              semg.at[b]).wait()

    def fire_scatter(b, m):
        pltpu.async_copy(wrows.at[b], acc.at[didx.at[m]], sems.at[b],
                         add=True)
        pltpu.async_copy(wcol.at[b], dacc.at[didx.at[m]], sems.at[b],
                         add=True)

    def drain_scatter(b, m):
        pltpu.make_async_copy(wrows.at[b], acc.at[didx.at[m]],
                              sems.at[b]).wait()
        pltpu.make_async_copy(wcol.at[b], dacc.at[didx.at[m]],
                              sems.at[b]).wait()

    def process(b):
        @plsc.parallel_loop(0, _CH, step=16, unroll=1)
        def _(r0):
            z = s1g[b, pl.ds(r0, 16)] + s2g[b, pl.ds(r0, 16)]
            e = jnp.where(z > 0, z, z * _ALPHA)
            w = jnp.exp(e)
            wcol[b, pl.ds(r0, 16)] = w
            wvs = [jnp.full((16,), w[rl], jnp.float32) for rl in range(16)]
            for jj in range(8):
                for rl in range(16):
                    wrows[b, r0 + rl, pl.ds(jj * 16, 16)] = (
                        rows[b, r0 + rl, pl.ds(jj * 16, 16)] * wvs[rl])

    pltpu.sync_copy(zero_hbm.at[pl.ds(s * _RPT, _RPT)],
                    acc.at[pl.ds(s * _RPT, _RPT)])
    pltpu.sync_copy(zerod_hbm.at[pl.ds(s * _RPT, _RPT)],
                    dacc.at[pl.ds(s * _RPT, _RPT)])
    fire_idx(0, 0, 0)
    fire_idx(1, 1, 1)
    wait_idx(0, 0)
    fire_gather(0, 0)
    plsc.subcore_barrier()

    def body(j, carry):
        for k in range(4):
            b = k & 1
            i = 4 * j + k
            wait_idx(1 - b, (k + 1) & 3)
            if k < 2:
                @pl.when(j > 0)
                def _():
                    drain_scatter(b, (k + 2) & 3)
            else:
                drain_scatter(b, (k + 2) & 3)
            if k == 3:
                @pl.when(j < _NJ - 1)
                def _():
                    fire_idx(i + 2, b, (k + 2) & 3)
            else:
                fire_idx(i + 2, b, (k + 2) & 3)
            fire_gather(1 - b, (k + 1) & 3)
            wait_gather(b, k)
            process(b)
            fire_scatter(b, k)
        return carry

    lax.fori_loop(0, _NJ, body, 0)

    drain_scatter(0, 2)
    wait_gather(0, 0)
    process(0)
    fire_scatter(0, 0)
    drain_scatter(1, 3)
    drain_scatter(0, 0)

    plsc.subcore_barrier()
    pltpu.sync_copy(acc.at[pl.ds(s * _RPT, _RPT)],
                    out_hbm.at[c, pl.ds(s * _RPT, _RPT)])
    pltpu.sync_copy(dacc.at[pl.ds(s * _RPT, _RPT)],
                    outd_hbm.at[c, pl.ds(s * _RPT, _RPT)])


_sc_call = functools.partial(
    pl.kernel,
    out_type=(jax.ShapeDtypeStruct((_NC, _NP, _F), jnp.float32),
              jax.ShapeDtypeStruct((_NC, _NP), jnp.float32)),
    mesh=plsc.VectorSubcoreMesh(core_axis_name="c", subcore_axis_name="s",
                                num_cores=_NC, num_subcores=_NS),
    scratch_types=[
        pltpu.VMEM((4, _CH), jnp.int32),
        pltpu.VMEM((4, _CH), jnp.int32),
        pltpu.VMEM((2, _CH), jnp.float32),
        pltpu.VMEM((2, _CH), jnp.float32),
        pltpu.VMEM((2, _CH, _F), jnp.float32),
        pltpu.VMEM((2, _CH, _F), jnp.float32),
        pltpu.VMEM((2, _CH), jnp.float32),
        pltpu.VMEM_SHARED((_NP, _F), jnp.float32),
        pltpu.VMEM_SHARED((_NP,), jnp.float32),
        pltpu.SemaphoreType.DMA((2,)),
        pltpu.SemaphoreType.DMA((2,)),
        pltpu.SemaphoreType.DMA((2,)),
    ],
    compiler_params=pltpu.CompilerParams(use_tc_tiling_on_sc=False,
                                         needs_layout_passes=False),
)(_sc_body)


def _tc2_body(acc_ref, dacc_ref, x_ref, wih_ref, whh_ref, bih_ref, bhh_ref,
              out_ref):
    h_msg = acc_ref[0] + acc_ref[1]
    denom = dacc_ref[0] + dacc_ref[1]
    h_prime = h_msg / jnp.maximum(denom, 1e-9)
    gi = jnp.dot(x_ref[...], wih_ref[...],
                 preferred_element_type=jnp.float32,
                 precision=lax.Precision.HIGHEST) + bih_ref[...]
    gh = jnp.dot(h_prime, whh_ref[...],
                 preferred_element_type=jnp.float32,
                 precision=lax.Precision.HIGHEST) + bhh_ref[...]
    r = jax.nn.sigmoid(gi[:, :_F] + gh[:, :_F])
    z = jax.nn.sigmoid(gi[:, _F:2 * _F] + gh[:, _F:2 * _F])
    n = jnp.tanh(gi[:, 2 * _F:] + r * gh[:, 2 * _F:])
    out_ref[...] = (1.0 - z) * n + z * h_prime


def _tc2(acc, dacc, x, wihT, whhT, bih, bhh):
    nb = 10
    rb = _N // nb
    return pl.pallas_call(
        _tc2_body,
        grid=(nb,),
        in_specs=[
            pl.BlockSpec((_NC, rb, _F), lambda i: (0, i, 0)),
            pl.BlockSpec((_NC, rb, 1), lambda i: (0, i, 0)),
            pl.BlockSpec((rb, _F), lambda i: (i, 0)),
            pl.BlockSpec((_F, 3 * _F), lambda i: (0, 0)),
            pl.BlockSpec((_F, 3 * _F), lambda i: (0, 0)),
            pl.BlockSpec((1, 3 * _F), lambda i: (0, 0)),
            pl.BlockSpec((1, 3 * _F), lambda i: (0, 0)),
        ],
        out_specs=pl.BlockSpec((rb, _F), lambda i: (i, 0)),
        out_shape=jax.ShapeDtypeStruct((_N, _F), jnp.float32),
    )(acc, dacc, x, wihT, whhT, bih, bhh)


def kernel(x, edge_index, W, a, W_ih, W_hh, b_ih, b_hh):
    src = edge_index[0]
    dst = edge_index[1]
    a1 = a[:_F]
    a2 = a[_F:]
    wh, s1, s2 = _tc1(x, W, a1, a2)
    zeros = jnp.zeros((_NP, _F), jnp.float32)
    zerosd = jnp.zeros((_NP,), jnp.float32)
    acc, dacc = _sc_call(src, dst, s1.reshape(_N), s2.reshape(_N), wh,
                         zeros, zerosd)
    return _tc2(acc, dacc.reshape(_NC, _NP, 1), x, W_ih.T, W_hh.T,
                b_ih.reshape(1, 3 * _F), b_hh.reshape(1, 3 * _F))
